# baseline (speedup 1.0000x reference)
import jax
import jax.numpy as jnp
from jax import lax
from jax.experimental import pallas as pl
from jax.experimental.pallas import tpu as pltpu
from jax.experimental.pallas import tpu_sc as plsc

B = 16384
D = 64
NNEG = 20
NW = 32
ROWS_PER_W = B // NW
GR = 128
NT = 4
NC = 4
NN = ROWS_PER_W * NNEG // GR
NJ = NT + NC + NN
NR = 8
BB = 1024


def _sc_body(pos_t, pos_c, nidx2d, temb, cemb, tout, cout, ntout,
             tidx, cidx, nidxT, bufs, *sems):
    gsem = sems[:NR]
    wsem = sems[NR:]
    wid = lax.axis_index("s") * 2 + lax.axis_index("c")
    base = wid * ROWS_PER_W
    nbase = wid * ROWS_PER_W * NNEG

    pltpu.sync_copy(pos_t.at[pl.ds(base, ROWS_PER_W)], tidx)
    pltpu.sync_copy(pos_c.at[pl.ds(base, ROWS_PER_W)], cidx)
    pltpu.sync_copy(nidx2d.at[pl.ds(wid * NN, NN)], nidxT)

    def buf(b):
        return bufs.at[pl.ds(b * GR, GR)]

    def g_issue(j, b):
        if isinstance(j, int) and j < NT:
            src = temb.at[tidx.at[pl.ds(j * GR, GR)]]
        elif isinstance(j, int) and j < NT + NC:
            src = cemb.at[cidx.at[pl.ds((j - NT) * GR, GR)]]
        else:
            src = cemb.at[nidxT.at[j - NT - NC]]
        pltpu.async_copy(src, buf(b), gsem[b])

    def g_drain(b):
        pltpu.make_async_copy(cemb.at[pl.ds(0, GR)], buf(b), gsem[b]).wait()

    def w_issue(j, b):
        if isinstance(j, int) and j < NT:
            dst = tout.at[pl.ds(base + j * GR, GR)]
        elif isinstance(j, int) and j < NT + NC:
            dst = cout.at[pl.ds(base + (j - NT) * GR, GR)]
        else:
            dst = ntout.at[pl.ds(nbase + (j - NT - NC) * GR, GR)]
        pltpu.async_copy(buf(b), dst, wsem[b])

    def w_drain(b):
        pltpu.make_async_copy(buf(b), ntout.at[pl.ds(0, GR)], wsem[b]).wait()

    for b in range(4):
        g_issue(b, b)

    for j in range(8):
        b = j
        if j >= 4:
            w_drain((b + 4) % NR)
        g_issue(j + 4, (b + 4) % NR)
        g_drain(b)
        w_issue(j, b)

    def steady(i, carry):
        for b in range(NR):
            j = i * NR + b
            w_drain((b + 4) % NR)
            pltpu.async_copy(cemb.at[nidxT.at[j - 4]],
                             buf((b + 4) % NR), gsem[(b + 4) % NR])
            g_drain(b)
            pltpu.async_copy(buf(b),
                             ntout.at[pl.ds(nbase + (j - NT - NC) * GR, GR)],
                             wsem[b])
        return carry
    lax.fori_loop(1, NJ // NR, steady, 0)

    for j in range(80, NJ):
        b = j % NR
        w_drain((b + 4) % NR)
        if j + 4 < NJ:
            g_issue(j + 4, (b + 4) % NR)
        g_drain(b)
        w_issue(j, b)
    for b in range(4, NR):
        w_drain(b)


_sc_gather = pl.kernel(
    _sc_body,
    out_type=[
        jax.ShapeDtypeStruct((B, D), jnp.float32),
        jax.ShapeDtypeStruct((B, D), jnp.float32),
        jax.ShapeDtypeStruct((B * NNEG, D), jnp.float32),
    ],
    mesh=plsc.VectorSubcoreMesh(core_axis_name="c", subcore_axis_name="s"),
    compiler_params=pltpu.CompilerParams(needs_layout_passes=False,
                                         use_tc_tiling_on_sc=False),
    scratch_types=[
        pltpu.VMEM((ROWS_PER_W,), jnp.int32),
        pltpu.VMEM((ROWS_PER_W,), jnp.int32),
        pltpu.VMEM((NN, GR), jnp.int32),
        pltpu.VMEM((NR * GR, D), jnp.float32),
    ] + [pltpu.SemaphoreType.DMA] * (2 * NR),
)


def _tc_body(t_ref, c_ref, n_ref, o_ref):
    bi = pl.program_id(0)
    k = pl.program_id(1)
    t = t_ref[:]
    nd = jnp.clip(jnp.sum(t * n_ref[:], axis=1), -10.0, 10.0)
    sneg = jnp.sum(jnp.log1p(jnp.exp(nd)))
    pd = jnp.clip(jnp.sum(t * c_ref[:], axis=1), -10.0, 10.0)
    spos = jnp.sum(jnp.log1p(jnp.exp(-pd)))
    val = sneg + jnp.where(k == 0, spos, 0.0)

    @pl.when((bi == 0) & (k == 0))
    def _():
        o_ref[0, 0] = 0.0

    o_ref[0, 0] += val * (1.0 / B)


_tc_loss = pl.pallas_call(
    _tc_body,
    grid=(B // BB, NNEG),
    in_specs=[
        pl.BlockSpec((BB, D), lambda bi, k: (bi, 0)),
        pl.BlockSpec((BB, D), lambda bi, k: (bi, 0)),
        pl.BlockSpec((BB, D), lambda bi, k: (k * (B // BB) + bi, 0)),
    ],
    out_specs=pl.BlockSpec((1, 1), lambda bi, k: (0, 0),
                           memory_space=pltpu.SMEM),
    out_shape=jax.ShapeDtypeStruct((1, 1), jnp.float32),
)


def kernel(pos_target, pos_context, neg_context, target_emb, context_emb):
    neg_t = neg_context.reshape(B, NNEG).T.reshape(B * NNEG // GR, GR)
    tout, cout, ntout = _sc_gather(pos_target, pos_context, neg_t,
                                   target_emb, context_emb)
    loss = _tc_loss(tout, cout, ntout)
    return loss[0, 0]


# speedup vs baseline: 1.6472x; 1.6472x over previous
import jax
import jax.numpy as jnp
from jax import lax
from jax.experimental import pallas as pl
from jax.experimental.pallas import tpu as pltpu
from jax.experimental.pallas import tpu_sc as plsc

B = 16384
D = 64
NNEG = 20
NSLOT = NNEG + 1
NW = 32
ROWS_PER_W = B // NW
GR = 128
NCH = ROWS_PER_W * NSLOT // GR
WPW = ROWS_PER_W * NSLOT
NRB = 4


def _sc_body(pos_t, cidx_hbm, temb, cemb, dots,
             tidx, cidx, tgtv, bufs, outv, tsem, *gsem):
    wid = lax.axis_index("s") * 2 + lax.axis_index("c")
    base = wid * ROWS_PER_W
    lane = lax.iota(jnp.int32, 16)

    pltpu.sync_copy(pos_t.at[pl.ds(base, ROWS_PER_W)], tidx)
    pltpu.sync_copy(cidx_hbm.at[pl.ds(wid * NCH, NCH)], cidx)

    for k in range(4):
        pltpu.async_copy(temb.at[tidx.at[pl.ds(k * GR, GR)]],
                         tgtv.at[pl.ds(k * GR, GR)], tsem)
    for m in range(NRB):
        pltpu.async_copy(cemb.at[cidx.at[m]],
                         bufs.at[pl.ds(m * GR, GR)], gsem[m])
    for k in range(4):
        pltpu.make_async_copy(temb.at[pl.ds(0, GR)],
                              tgtv.at[pl.ds(k * GR, GR)], tsem).wait()

    def drain(b):
        pltpu.make_async_copy(cemb.at[pl.ds(0, GR)],
                              bufs.at[pl.ds(b * GR, GR)], gsem[b]).wait()

    def chunk_compute(j, b, rs):
        def grp_body(i2, rs2):
            r, slot = rs2
            acc = jnp.zeros((16,), jnp.float32)
            for k in range(16):
                i = i2 * 16 + k
                t0 = tgtv[r, pl.ds(0, 16)]
                t1 = tgtv[r, pl.ds(16, 16)]
                t2 = tgtv[r, pl.ds(32, 16)]
                t3 = tgtv[r, pl.ds(48, 16)]
                cr = b * GR + i
                v = (t0 * bufs[cr, pl.ds(0, 16)]
                     + t1 * bufs[cr, pl.ds(16, 16)]
                     + t2 * bufs[cr, pl.ds(32, 16)]
                     + t3 * bufs[cr, pl.ds(48, 16)])
                acc = jnp.where(lane == k, jnp.sum(v), acc)
                ns = slot + 1
                wrap = ns >= NSLOT
                r = jnp.where(wrap, r + 1, r)
                slot = jnp.where(wrap, 0, ns)
            outv[pl.ds(j * GR + i2 * 16, 16)] = acc
            return (r, slot)
        return lax.fori_loop(0, GR // 16, grp_body, rs)

    rs0 = (jnp.int32(0), jnp.int32(0))

    def ring_body(j4, rs):
        for b in range(NRB):
            j = j4 * NRB + b
            drain(b)
            rs = chunk_compute(j, b, rs)
            pltpu.async_copy(cemb.at[cidx.at[j + NRB]],
                             bufs.at[pl.ds(b * GR, GR)], gsem[b])
        return rs
    rs = lax.fori_loop(0, NCH // NRB - 1, ring_body, rs0)
    for j in range(NCH - NRB, NCH):
        b = j % NRB
        drain(b)
        rs = chunk_compute(j, b, rs)

    pltpu.sync_copy(outv, dots.at[pl.ds(wid * WPW, WPW)])


_sc_dots = pl.kernel(
    _sc_body,
    out_type=jax.ShapeDtypeStruct((B * NSLOT,), jnp.float32),
    mesh=plsc.VectorSubcoreMesh(core_axis_name="c", subcore_axis_name="s"),
    compiler_params=pltpu.CompilerParams(needs_layout_passes=False,
                                         use_tc_tiling_on_sc=False),
    scratch_types=[
        pltpu.VMEM((ROWS_PER_W,), jnp.int32),
        pltpu.VMEM((NCH, GR), jnp.int32),
        pltpu.VMEM((ROWS_PER_W, D), jnp.float32),
        pltpu.VMEM((NRB * GR, D), jnp.float32),
        pltpu.VMEM((WPW,), jnp.float32),
    ] + [pltpu.SemaphoreType.DMA] * (1 + NRB),
)


def _tc_loss_body(d_ref, o_ref):
    x = d_ref[:]
    g = (lax.broadcasted_iota(jnp.int32, x.shape, 0) * 128
         + lax.broadcasted_iota(jnp.int32, x.shape, 1))
    slot = g % NSLOT
    xc = jnp.clip(x, -10.0, 10.0)
    pos_f = jnp.log1p(jnp.exp(-xc))
    neg_f = jnp.log1p(jnp.exp(xc))
    contrib = jnp.where(slot == 0, pos_f, neg_f)
    o_ref[0, 0] = jnp.sum(contrib) * (1.0 / B)


_tc_loss = pl.pallas_call(
    _tc_loss_body,
    out_shape=jax.ShapeDtypeStruct((1, 1), jnp.float32),
    in_specs=[pl.BlockSpec(memory_space=pltpu.VMEM)],
    out_specs=pl.BlockSpec(memory_space=pltpu.SMEM),
)


def kernel(pos_target, pos_context, neg_context, target_emb, context_emb):
    cidx = jnp.concatenate([pos_context[:, None], neg_context], axis=1)
    cidx_hbm = cidx.reshape(B * NSLOT // GR, GR)
    dots = _sc_dots(pos_target, cidx_hbm, target_emb, context_emb)
    loss = _tc_loss(dots.reshape(B * NSLOT // GR, GR))
    return loss[0, 0]
